# baseline (speedup 1.0000x reference)
import jax
import jax.numpy as jnp
from jax.experimental import pallas as pl

N_TOKENS = 32768
D_MODEL = 768
N_EXPERTS = 8
TEMPERATURE = 1.0

BLK = 512


def _router_block(x_ref, wt_ref, out_ref):
    xb = x_ref[...].astype(jnp.bfloat16)
    out_ref[...] = jnp.dot(xb, wt_ref[...], preferred_element_type=jnp.float32)


def kernel(x, W):
    n_tokens, d_model = x.shape
    n_experts = W.shape[0]
    wt = W.T.astype(jnp.bfloat16)

    grid = (n_tokens // BLK,)
    logits = pl.pallas_call(
        _router_block,
        grid=grid,
        in_specs=[
            pl.BlockSpec((BLK, d_model), lambda i: (i, 0)),
            pl.BlockSpec((d_model, n_experts), lambda i: (0, 0)),
        ],
        out_specs=pl.BlockSpec((BLK, n_experts), lambda i: (i, 0)),
        out_shape=jax.ShapeDtypeStruct((n_tokens, n_experts), jnp.float32),
    )(x, wt)

    temp = max(TEMPERATURE, 1e-06)
    if temp != 1.0:
        logits = logits / temp
    return logits


# speedup vs baseline: 1.5869x; 1.5869x over previous
import jax
import jax.numpy as jnp
from jax.experimental import pallas as pl

N_TOKENS = 32768
D_MODEL = 768
N_EXPERTS = 8
TEMPERATURE = 1.0

BLK = 4096


def _router_block(x_ref, wt_ref, out_ref):
    xb = x_ref[...].astype(jnp.bfloat16)
    out_ref[...] = jnp.dot(xb, wt_ref[...], preferred_element_type=jnp.float32)


def kernel(x, W):
    n_tokens, d_model = x.shape
    n_experts = W.shape[0]
    wt = W.T.astype(jnp.bfloat16)

    grid = (n_tokens // BLK,)
    logits = pl.pallas_call(
        _router_block,
        grid=grid,
        in_specs=[
            pl.BlockSpec((BLK, d_model), lambda i: (i, 0)),
            pl.BlockSpec((d_model, n_experts), lambda i: (0, 0)),
        ],
        out_specs=pl.BlockSpec((BLK, n_experts), lambda i: (i, 0)),
        out_shape=jax.ShapeDtypeStruct((n_tokens, n_experts), jnp.float32),
    )(x, wt)

    temp = max(TEMPERATURE, 1e-06)
    if temp != 1.0:
        logits = logits / temp
    return logits
